# Initial kernel scaffold; baseline (speedup 1.0000x reference)
import jax
import jax.numpy as jnp
from jax.experimental import pallas as pl


def kernel(token_id, embed_weight):
    raise NotImplementedError("write your pallas kernel here")


# speedup vs baseline: 1.4502x; 1.4502x over previous
import jax
import jax.numpy as jnp
from jax import lax
from jax.experimental import pallas as pl
from jax.experimental.pallas import tpu as pltpu
from jax.experimental.pallas import tpu_sc as plsc

_NC = 2
_NS = 16
_NW = _NC * _NS
_L = 16

_VOCAB = 160
_D = 1024
_B = 4 * 8192
_BPW = _B // _NW
_C = 32
_NCHUNK = _BPW // _C


def _emb_body(idx_hbm, table_hbm, out_hbm, mask_hbm, idx_v, mask_v, rows_v, sem):
    wid = lax.axis_index("s") * _NC + lax.axis_index("c")
    base = wid * _BPW
    pltpu.sync_copy(idx_hbm.at[wid], idx_v)

    def chunk(c, carry):
        pltpu.async_copy(table_hbm.at[idx_v.at[c]], rows_v, sem).wait()
        pltpu.sync_copy(rows_v, out_hbm.at[pl.ds(base + c * _C, _C)])
        for j in range(_C // _L):
            v = idx_v[c, pl.ds(j * _L, _L)]
            mask_v[c, pl.ds(j * _L, _L)] = jnp.where(
                v == 0, jnp.int32(1), jnp.int32(0))
        return carry

    lax.fori_loop(0, _NCHUNK, chunk, 0)
    pltpu.sync_copy(mask_v, mask_hbm.at[wid])


def kernel(token_id, embed_weight):
    tid = token_id.astype(jnp.int32)
    idx = tid.reshape(_NW, _NCHUNK, _C)
    mesh = plsc.VectorSubcoreMesh(core_axis_name="c", subcore_axis_name="s")
    out, mask = pl.kernel(
        _emb_body,
        out_type=[
            jax.ShapeDtypeStruct((_B, _D), jnp.float32),
            jax.ShapeDtypeStruct((_NW, _NCHUNK, _C), jnp.int32),
        ],
        mesh=mesh,
        scratch_types=[
            pltpu.VMEM((_NCHUNK, _C), jnp.int32),
            pltpu.VMEM((_NCHUNK, _C), jnp.int32),
            pltpu.VMEM((_C, _D), jnp.float32),
            pltpu.SemaphoreType.DMA,
        ],
    )(idx, embed_weight)
    x = out.reshape(token_id.shape[0], token_id.shape[1], _D)
    padding_mask = mask.reshape(token_id.shape).astype(bool)
    return (x, padding_mask, token_id)


# speedup vs baseline: 1.5067x; 1.0390x over previous
import jax
import jax.numpy as jnp
from jax import lax
from jax.experimental import pallas as pl
from jax.experimental.pallas import tpu as pltpu
from jax.experimental.pallas import tpu_sc as plsc

_NC = 2
_NS = 16
_NW = _NC * _NS
_L = 16

_VOCAB = 160
_D = 1024
_B = 4 * 8192
_BPW = _B // _NW
_C = 32
_NCHUNK = _BPW // _C


def _emb_body(idx_hbm, table_hbm, out_hbm, mask_hbm,
              idx_v, mask_v, rows_a, rows_b,
              gsem, osem_a, osem_b):
    sid = lax.axis_index("s")
    wid = sid * _NC + lax.axis_index("c")
    base = wid * _BPW

    pltpu.sync_copy(idx_hbm.at[wid], idx_v)

    bufs = (rows_a, rows_b)
    osems = (osem_a, osem_b)

    for b in range(2):
        pltpu.async_copy(table_hbm.at[idx_v.at[b]], bufs[b], gsem).wait()
        pltpu.async_copy(bufs[b], out_hbm.at[pl.ds(base + b * _C, _C)],
                         osems[b])

    def pair(g, carry):
        for b in range(2):
            c = 2 * g + b
            pltpu.make_async_copy(bufs[b], out_hbm.at[pl.ds(base, _C)],
                                  osems[b]).wait()
            pltpu.async_copy(table_hbm.at[idx_v.at[c]], bufs[b], gsem).wait()
            pltpu.async_copy(bufs[b], out_hbm.at[pl.ds(base + c * _C, _C)],
                             osems[b])
        return carry

    lax.fori_loop(1, _NCHUNK // 2, pair, 0)

    def mrow(c, carry):
        for j in range(_C // _L):
            v = idx_v[c, pl.ds(j * _L, _L)]
            mask_v[c, pl.ds(j * _L, _L)] = jnp.where(
                v == 0, jnp.int32(1), jnp.int32(0))
        return carry

    lax.fori_loop(0, _NCHUNK, mrow, 0)
    pltpu.sync_copy(mask_v, mask_hbm.at[wid])

    for b in range(2):
        pltpu.make_async_copy(bufs[b], out_hbm.at[pl.ds(base, _C)],
                              osems[b]).wait()


def kernel(token_id, embed_weight):
    tid = token_id.astype(jnp.int32)
    idx = tid.reshape(_NW, _NCHUNK, _C)
    mesh = plsc.VectorSubcoreMesh(core_axis_name="c", subcore_axis_name="s")
    out, mask = pl.kernel(
        _emb_body,
        out_type=[
            jax.ShapeDtypeStruct((_B, _D), jnp.float32),
            jax.ShapeDtypeStruct((_NW, _NCHUNK, _C), jnp.int32),
        ],
        mesh=mesh,
        scratch_types=[
            pltpu.VMEM((_NCHUNK, _C), jnp.int32),
            pltpu.VMEM((_NCHUNK, _C), jnp.int32),
            pltpu.VMEM((_C, _D), jnp.float32),
            pltpu.VMEM((_C, _D), jnp.float32),
            pltpu.SemaphoreType.DMA,
            pltpu.SemaphoreType.DMA,
            pltpu.SemaphoreType.DMA,
        ],
    )(idx, embed_weight)
    x = out.reshape(token_id.shape[0], token_id.shape[1], _D)
    padding_mask = mask.reshape(token_id.shape).astype(bool)
    return (x, padding_mask, token_id)
